# Initial kernel scaffold; baseline (speedup 1.0000x reference)
import jax
import jax.numpy as jnp
from jax.experimental import pallas as pl


def kernel(input_tensor, h_cur, c_cur, L_vals, weight, bias, L_rows, L_cols):
    raise NotImplementedError("write your pallas kernel here")


# speedup vs baseline: 2.8086x; 2.8086x over previous
import functools
import jax
import jax.numpy as jnp
from jax import lax
from jax.experimental import pallas as pl
from jax.experimental.pallas import tpu as pltpu
from jax.experimental.pallas import tpu_sc as plsc

MV = 10000
MPAD = 10240
NBATCH = 2
FTOT = 64
CHEB_K = 3
FW = FTOT * NBATCH
GOUT = 4 * 32
NNZV = 320000

NCORE = 2
NSUB = 16
NWORK = NCORE * NSUB
ECHUNK = 128
NCHUNK = 80
EPW = ECHUNK * NCHUNK
NNZ_PAD = EPW * NWORK
RPT = MPAD // NSUB
RSUB = 128


def _spmm_body(x_hbm, rows_hbm, cols_hbm, vals_hbm, out_hbm,
               rows_v, cols_v, vals_v, gbuf, acc, sem):
    cid = lax.axis_index("c")
    sid = lax.axis_index("s")
    wid = sid * NCORE + cid

    pltpu.sync_copy(rows_hbm.at[wid], rows_v)
    pltpu.sync_copy(cols_hbm.at[wid], cols_v)
    pltpu.sync_copy(vals_hbm.at[wid], vals_v)

    def zbody(i, _):
        for cg in range(FW // 16):
            gbuf[i, pl.ds(cg * 16, 16)] = jnp.zeros((16,), jnp.float32)
        return 0
    lax.fori_loop(0, RSUB, zbody, 0, unroll=4)
    for i in range(RPT // RSUB):
        pltpu.sync_copy(gbuf, acc.at[pl.ds(sid * RPT + i * RSUB, RSUB)])
    plsc.subcore_barrier()

    def chunk(j, _):
        pltpu.async_copy(x_hbm.at[cols_v.at[j]], gbuf, sem).wait()

        def ebody(g, _):
            vv = vals_v[j, pl.ds(g * 16, 16)]
            for t in range(16):
                val = vv[t]
                e = g * 16 + t
                for cg in range(FW // 16):
                    sl = pl.ds(cg * 16, 16)
                    gbuf[e, sl] = gbuf[e, sl] * val
            return 0
        lax.fori_loop(0, ECHUNK // 16, ebody, 0)
        pltpu.sync_copy(gbuf, acc.at[rows_v.at[j]], add=True)
        return 0
    lax.fori_loop(0, NCHUNK, chunk, 0)
    plsc.subcore_barrier()

    for i in range(RPT // RSUB):
        base = sid * RPT + i * RSUB
        pltpu.sync_copy(acc.at[pl.ds(base, RSUB)], gbuf)
        pltpu.sync_copy(gbuf, out_hbm.at[cid, pl.ds(base, RSUB)])


_spmm_partial = functools.partial(
    pl.kernel,
    mesh=plsc.VectorSubcoreMesh(core_axis_name="c", subcore_axis_name="s"),
    out_type=jax.ShapeDtypeStruct((NCORE, MPAD, FW), jnp.float32),
    scratch_types=[
        pltpu.VMEM((NCHUNK, ECHUNK), jnp.int32),
        pltpu.VMEM((NCHUNK, ECHUNK), jnp.int32),
        pltpu.VMEM((NCHUNK, ECHUNK), jnp.float32),
        pltpu.VMEM((ECHUNK, FW), jnp.float32),
        pltpu.VMEM_SHARED((MPAD, FW), jnp.float32),
        pltpu.SemaphoreType.DMA,
    ],
)(_spmm_body)


MB = 1024


def _combine_body(p_ref, a0_ref, o_ref):
    o_ref[...] = p_ref[0] + p_ref[1] - a0_ref[...]


def _combine(partials, a0):
    return pl.pallas_call(
        _combine_body,
        grid=(MPAD // MB,),
        in_specs=[
            pl.BlockSpec((NCORE, MB, FW), lambda i: (0, i, 0)),
            pl.BlockSpec((MB, FW), lambda i: (i, 0)),
        ],
        out_specs=pl.BlockSpec((MB, FW), lambda i: (i, 0)),
        out_shape=jax.ShapeDtypeStruct((MPAD, FW), jnp.float32),
    )(partials, a0)


def _matmul_body(q_ref, x1_ref, a0_ref, w_ref, o_ref):
    a0 = a0_ref[...]
    x1 = x1_ref[...]
    t2 = q_ref[0] + q_ref[1] - x1
    x2 = 2.0 * t2 - a0
    w = w_ref[...]
    for n in range(NBATCH):
        o_ref[n, :, :] = (
            jnp.dot(a0, w[0, n], preferred_element_type=jnp.float32)
            + jnp.dot(x1, w[1, n], preferred_element_type=jnp.float32)
            + jnp.dot(x2, w[2, n], preferred_element_type=jnp.float32)
        )


MMB = 1000


def _matmul(partials2, x1, a0, wn):
    return pl.pallas_call(
        _matmul_body,
        grid=(MV // MMB,),
        in_specs=[
            pl.BlockSpec((NCORE, MMB, FW), lambda i: (0, i, 0)),
            pl.BlockSpec((MMB, FW), lambda i: (i, 0)),
            pl.BlockSpec((MMB, FW), lambda i: (i, 0)),
            pl.BlockSpec((CHEB_K, NBATCH, FW, GOUT), lambda i: (0, 0, 0, 0)),
        ],
        out_specs=pl.BlockSpec((NBATCH, MMB, GOUT), lambda i: (0, i, 0)),
        out_shape=jax.ShapeDtypeStruct((NBATCH, MV, GOUT), jnp.float32),
    )(partials2, x1, a0, wn)


JB = 10000


def _gates_body(cv_ref, b_ref, c_ref, h_ref, cn_ref):
    cv = cv_ref[...] + b_ref[...][None, :, :]
    gi = jax.nn.sigmoid(cv[:, 0:32])
    gf = jax.nn.sigmoid(cv[:, 32:64])
    go = jax.nn.sigmoid(cv[:, 64:96])
    gg = jnp.tanh(cv[:, 96:128])
    cn = gf * c_ref[...] + gi * gg
    cn_ref[...] = cn
    h_ref[...] = go * jnp.tanh(cn)


def _gates(conv, bias_b, c_cur):
    return pl.pallas_call(
        _gates_body,
        grid=(1,),
        in_specs=[
            pl.BlockSpec((NBATCH, GOUT, JB), lambda i: (0, 0, 0)),
            pl.BlockSpec((GOUT, JB), lambda i: (0, 0)),
            pl.BlockSpec((NBATCH, 32, JB), lambda i: (0, 0, 0)),
        ],
        out_specs=[
            pl.BlockSpec((NBATCH, 32, JB), lambda i: (0, 0, 0)),
            pl.BlockSpec((NBATCH, 32, JB), lambda i: (0, 0, 0)),
        ],
        out_shape=[
            jax.ShapeDtypeStruct((NBATCH, 32, MV), jnp.float32),
            jax.ShapeDtypeStruct((NBATCH, 32, MV), jnp.float32),
        ],
    )(conv, bias_b, c_cur)


def kernel(input_tensor, h_cur, c_cur, L_vals, weight, bias, L_rows, L_cols):
    h_cur = h_cur.astype(jnp.float32)
    c_cur = c_cur.astype(jnp.float32)

    combined = jnp.concatenate([input_tensor, h_cur], axis=1)
    a0 = jnp.transpose(combined, (2, 1, 0)).reshape(MV, FW)
    a0 = jnp.pad(a0, ((0, MPAD - MV), (0, 0)))

    pad = NNZ_PAD - NNZV
    rows3 = jnp.concatenate(
        [L_rows, jnp.zeros((pad,), jnp.int32)]).reshape(NWORK, NCHUNK, ECHUNK)
    cols3 = jnp.concatenate(
        [L_cols, jnp.zeros((pad,), jnp.int32)]).reshape(NWORK, NCHUNK, ECHUNK)
    vals3 = jnp.concatenate(
        [L_vals, jnp.zeros((pad,), jnp.float32)]).reshape(NWORK, NCHUNK, ECHUNK)

    wr = weight.reshape(FTOT, CHEB_K, GOUT).transpose(1, 0, 2)
    wz = jnp.zeros_like(wr)
    wn0 = jnp.stack([wr, wz], axis=2).reshape(CHEB_K, FW, GOUT)
    wn1 = jnp.stack([wz, wr], axis=2).reshape(CHEB_K, FW, GOUT)
    wn = jnp.stack([wn0, wn1], axis=1)

    p1 = _spmm_partial(a0, rows3, cols3, vals3)
    x1 = _combine(p1, a0)
    p2 = _spmm_partial(x1, rows3, cols3, vals3)
    mat = _matmul(p2, x1, a0, wn)

    conv = mat.reshape(NBATCH, GOUT, MV)
    bias_b = jnp.broadcast_to(bias.reshape(GOUT, 1), (GOUT, JB))
    h_next, c_next = _gates(conv, bias_b, c_cur)
    return (h_next, c_next)


# speedup vs baseline: 3.3335x; 1.1869x over previous
import functools
import jax
import jax.numpy as jnp
from jax import lax
from jax.experimental import pallas as pl
from jax.experimental.pallas import tpu as pltpu
from jax.experimental.pallas import tpu_sc as plsc

MV = 10000
MPAD = 10240
NBATCH = 2
FTOT = 64
CHEB_K = 3
FW = FTOT * NBATCH
GOUT = 4 * 32
NNZV = 320000

NCORE = 2
NSUB = 16
NWORK = NCORE * NSUB
ECHUNK = 64
NCHUNK = 160
EPW = ECHUNK * NCHUNK
NNZ_PAD = EPW * NWORK
NSLAB = 157


def _spmm_body(x_hbm, pk_hbm, out_hbm,
               ib, gbuf0, gbuf1, acc,
               is0, is1, is2, is3, gs0, gs1):
    cid = lax.axis_index("c")
    sid = lax.axis_index("s")
    wid = sid * NCORE + cid
    isem = (is0, is1, is2, is3)
    gsem = (gs0, gs1)
    gbufs = (gbuf0, gbuf1)

    def zbody(i, _):
        for cg in range(FW // 16):
            gbuf0[i, pl.ds(cg * 16, 16)] = jnp.zeros((16,), jnp.float32)
        return 0
    lax.fori_loop(0, ECHUNK, zbody, 0, unroll=4)
    for t in range(10):
        k = sid + NSUB * t
        if t < 9:
            pltpu.sync_copy(gbuf0, acc.at[pl.ds(k * 64, 64)])
        else:
            @pl.when(k < NSLAB - 1)
            def _():
                pltpu.sync_copy(gbuf0, acc.at[pl.ds(k * 64, 64)])
            @pl.when(k == NSLAB - 1)
            def _():
                pltpu.sync_copy(gbuf0.at[pl.ds(0, 16)],
                                acc.at[pl.ds(k * 64, 16)])
    plsc.subcore_barrier()

    def scale(r, gbuf):
        def ebody(g, _):
            vv = lax.bitcast_convert_type(ib[r, 2, pl.ds(g * 16, 16)], jnp.float32)
            for t in range(16):
                val = vv[t]
                e = g * 16 + t
                for cg in range(FW // 16):
                    sl = pl.ds(cg * 16, 16)
                    gbuf[e, sl] = gbuf[e, sl] * val
            return 0
        lax.fori_loop(0, ECHUNK // 16, ebody, 0)

    pltpu.sync_copy(pk_hbm.at[wid, 0], ib.at[0])
    pltpu.sync_copy(pk_hbm.at[wid, 1], ib.at[1])
    pltpu.async_copy(pk_hbm.at[wid, 2], ib.at[2], isem[2])
    pltpu.async_copy(pk_hbm.at[wid, 3], ib.at[3], isem[3])
    pltpu.async_copy(x_hbm.at[ib.at[0, 1]], gbuf0, gsem[0])
    pltpu.async_copy(x_hbm.at[ib.at[1, 1]], gbuf1, gsem[1])

    def quad(q, _):
        j_base = 4 * q
        for u in range(4):
            j = j_base + u
            b = u % 2
            r = u
            rn = (u + 2) % 4
            pltpu.make_async_copy(x_hbm.at[ib.at[r, 1]], gbufs[b],
                                  gsem[b]).wait()
            scale(r, gbufs[b])
            pltpu.sync_copy(gbufs[b], acc.at[ib.at[r, 0]], add=True)

            @pl.when(j + 4 < NCHUNK)
            def _():
                pltpu.async_copy(pk_hbm.at[wid, j + 4], ib.at[r], isem[r])

            @pl.when(j + 2 < NCHUNK)
            def _():
                pltpu.make_async_copy(pk_hbm.at[wid, j + 2], ib.at[rn],
                                      isem[rn]).wait()
                pltpu.async_copy(x_hbm.at[ib.at[rn, 1]], gbufs[b], gsem[b])
        return 0
    lax.fori_loop(0, NCHUNK // 4, quad, 0)
    plsc.subcore_barrier()

    for t in range(10):
        k = sid + NSUB * t
        if t < 9:
            pltpu.sync_copy(acc.at[pl.ds(k * 64, 64)], gbuf0)
            pltpu.sync_copy(gbuf0, out_hbm.at[cid, pl.ds(k * 64, 64)])
        else:
            @pl.when(k < NSLAB - 1)
            def _():
                pltpu.sync_copy(acc.at[pl.ds(k * 64, 64)], gbuf0)
                pltpu.sync_copy(gbuf0, out_hbm.at[cid, pl.ds(k * 64, 64)])
            @pl.when(k == NSLAB - 1)
            def _():
                pltpu.sync_copy(acc.at[pl.ds(k * 64, 16)],
                                gbuf0.at[pl.ds(0, 16)])
                pltpu.sync_copy(gbuf0.at[pl.ds(0, 16)],
                                out_hbm.at[cid, pl.ds(k * 64, 16)])


_spmm_partial = functools.partial(
    pl.kernel,
    mesh=plsc.VectorSubcoreMesh(core_axis_name="c", subcore_axis_name="s"),
    out_type=jax.ShapeDtypeStruct((NCORE, MPAD, FW), jnp.float32),
    scratch_types=[
        pltpu.VMEM((4, 3, ECHUNK), jnp.int32),
        pltpu.VMEM((ECHUNK, FW), jnp.float32),
        pltpu.VMEM((ECHUNK, FW), jnp.float32),
        pltpu.VMEM_SHARED((MV, FW), jnp.float32),
        pltpu.SemaphoreType.DMA,
        pltpu.SemaphoreType.DMA,
        pltpu.SemaphoreType.DMA,
        pltpu.SemaphoreType.DMA,
        pltpu.SemaphoreType.DMA,
        pltpu.SemaphoreType.DMA,
    ],
)(_spmm_body)


MB = 1024


def _combine_body(p_ref, a0_ref, o_ref):
    o_ref[...] = p_ref[0] + p_ref[1] - a0_ref[...]


def _combine(partials, a0):
    return pl.pallas_call(
        _combine_body,
        grid=(MPAD // MB,),
        in_specs=[
            pl.BlockSpec((NCORE, MB, FW), lambda i: (0, i, 0)),
            pl.BlockSpec((MB, FW), lambda i: (i, 0)),
        ],
        out_specs=pl.BlockSpec((MB, FW), lambda i: (i, 0)),
        out_shape=jax.ShapeDtypeStruct((MPAD, FW), jnp.float32),
    )(partials, a0)


def _matmul_body(q_ref, x1_ref, a0_ref, w_ref, o_ref):
    a0 = a0_ref[...]
    x1 = x1_ref[...]
    t2 = q_ref[0] + q_ref[1] - x1
    x2 = 2.0 * t2 - a0
    w = w_ref[...]
    for n in range(NBATCH):
        o_ref[n, :, :] = (
            jnp.dot(a0, w[0, n], preferred_element_type=jnp.float32)
            + jnp.dot(x1, w[1, n], preferred_element_type=jnp.float32)
            + jnp.dot(x2, w[2, n], preferred_element_type=jnp.float32)
        )


MMB = 1000


def _matmul(partials2, x1, a0, wn):
    return pl.pallas_call(
        _matmul_body,
        grid=(MV // MMB,),
        in_specs=[
            pl.BlockSpec((NCORE, MMB, FW), lambda i: (0, i, 0)),
            pl.BlockSpec((MMB, FW), lambda i: (i, 0)),
            pl.BlockSpec((MMB, FW), lambda i: (i, 0)),
            pl.BlockSpec((CHEB_K, NBATCH, FW, GOUT), lambda i: (0, 0, 0, 0)),
        ],
        out_specs=pl.BlockSpec((NBATCH, MMB, GOUT), lambda i: (0, i, 0)),
        out_shape=jax.ShapeDtypeStruct((NBATCH, MV, GOUT), jnp.float32),
    )(partials2, x1, a0, wn)


JB = 10000


def _gates_body(cv_ref, b_ref, c_ref, h_ref, cn_ref):
    cv = cv_ref[...] + b_ref[...][None, :, :]
    gi = jax.nn.sigmoid(cv[:, 0:32])
    gf = jax.nn.sigmoid(cv[:, 32:64])
    go = jax.nn.sigmoid(cv[:, 64:96])
    gg = jnp.tanh(cv[:, 96:128])
    cn = gf * c_ref[...] + gi * gg
    cn_ref[...] = cn
    h_ref[...] = go * jnp.tanh(cn)


def _gates(conv, bias_b, c_cur):
    return pl.pallas_call(
        _gates_body,
        grid=(1,),
        in_specs=[
            pl.BlockSpec((NBATCH, GOUT, JB), lambda i: (0, 0, 0)),
            pl.BlockSpec((GOUT, JB), lambda i: (0, 0)),
            pl.BlockSpec((NBATCH, 32, JB), lambda i: (0, 0, 0)),
        ],
        out_specs=[
            pl.BlockSpec((NBATCH, 32, JB), lambda i: (0, 0, 0)),
            pl.BlockSpec((NBATCH, 32, JB), lambda i: (0, 0, 0)),
        ],
        out_shape=[
            jax.ShapeDtypeStruct((NBATCH, 32, MV), jnp.float32),
            jax.ShapeDtypeStruct((NBATCH, 32, MV), jnp.float32),
        ],
    )(conv, bias_b, c_cur)


def kernel(input_tensor, h_cur, c_cur, L_vals, weight, bias, L_rows, L_cols):
    h_cur = h_cur.astype(jnp.float32)
    c_cur = c_cur.astype(jnp.float32)

    combined = jnp.concatenate([input_tensor, h_cur], axis=1)
    a0 = jnp.transpose(combined, (2, 1, 0)).reshape(MV, FW)
    a0 = jnp.pad(a0, ((0, MPAD - MV), (0, 0)))

    pad = NNZ_PAD - NNZV
    rows3 = jnp.concatenate(
        [L_rows, jnp.zeros((pad,), jnp.int32)]).reshape(NWORK, NCHUNK, ECHUNK)
    cols3 = jnp.concatenate(
        [L_cols, jnp.zeros((pad,), jnp.int32)]).reshape(NWORK, NCHUNK, ECHUNK)
    vbits3 = lax.bitcast_convert_type(jnp.concatenate(
        [L_vals, jnp.zeros((pad,), jnp.float32)]), jnp.int32
        ).reshape(NWORK, NCHUNK, ECHUNK)
    pk = jnp.stack([rows3, cols3, vbits3], axis=2)

    wr = weight.reshape(FTOT, CHEB_K, GOUT).transpose(1, 0, 2)
    wz = jnp.zeros_like(wr)
    wn0 = jnp.stack([wr, wz], axis=2).reshape(CHEB_K, FW, GOUT)
    wn1 = jnp.stack([wz, wr], axis=2).reshape(CHEB_K, FW, GOUT)
    wn = jnp.stack([wn0, wn1], axis=1)

    p1 = _spmm_partial(a0, pk)
    x1 = _combine(p1, a0)
    p2 = _spmm_partial(x1, pk)
    mat = _matmul(p2, x1, a0, wn)

    conv = mat.reshape(NBATCH, GOUT, MV)
    bias_b = jnp.broadcast_to(bias.reshape(GOUT, 1), (GOUT, JB))
    h_next, c_next = _gates(conv, bias_b, c_cur)
    return (h_next, c_next)


# speedup vs baseline: 3.9222x; 1.1766x over previous
import functools
import jax
import jax.numpy as jnp
from jax import lax
from jax.experimental import pallas as pl
from jax.experimental.pallas import tpu as pltpu
from jax.experimental.pallas import tpu_sc as plsc

MV = 10000
MPAD = 10240
NBATCH = 2
FTOT = 64
CHEB_K = 3
FW = FTOT * NBATCH
GOUT = 4 * 32
NNZV = 320000

NCORE = 2
NSUB = 16
NWORK = NCORE * NSUB
ECHUNK = 64
NCHUNK = 160
EPW = ECHUNK * NCHUNK
NNZ_PAD = EPW * NWORK
NSLAB = 157
CPT = 2 * NCHUNK
C0 = 232


def _spmm_body(x_hbm, pk_hbm, out_hbm,
               ib, gbuf0, gbuf1, acc,
               is0, is1, is2, is3, gs0, gs1):
    cid = lax.axis_index("c")
    sid = lax.axis_index("s")
    isem = (is0, is1, is2, is3)
    gsem = (gs0, gs1)
    gbufs = (gbuf0, gbuf1)
    base = sid * CPT + cid * C0
    cnt = jnp.where(cid == 0, C0, CPT - C0)
    nq = cnt // 4

    def zbody(i, _):
        for cg in range(FW // 16):
            gbuf0[i, pl.ds(cg * 16, 16)] = jnp.zeros((16,), jnp.float32)
        return 0
    lax.fori_loop(0, ECHUNK, zbody, 0, unroll=4)
    for t in range(10):
        k = sid + NSUB * t
        if t < 9:
            pltpu.sync_copy(gbuf0, acc.at[pl.ds(k * 64, 64)])
        else:
            @pl.when(k < NSLAB - 1)
            def _():
                pltpu.sync_copy(gbuf0, acc.at[pl.ds(k * 64, 64)])
            @pl.when(k == NSLAB - 1)
            def _():
                pltpu.sync_copy(gbuf0.at[pl.ds(0, 16)],
                                acc.at[pl.ds(k * 64, 16)])
    plsc.subcore_barrier()

    def scale(r, gbuf):
        def ebody(g, _):
            vv = lax.bitcast_convert_type(ib[r, 2, pl.ds(g * 16, 16)], jnp.float32)
            for t in range(16):
                val = vv[t]
                e = g * 16 + t
                for cg in range(FW // 16):
                    sl = pl.ds(cg * 16, 16)
                    gbuf[e, sl] = gbuf[e, sl] * val
            return 0
        lax.fori_loop(0, ECHUNK // 16, ebody, 0)

    pltpu.sync_copy(pk_hbm.at[base + 0], ib.at[0])
    pltpu.sync_copy(pk_hbm.at[base + 1], ib.at[1])
    pltpu.async_copy(pk_hbm.at[base + 2], ib.at[2], isem[2])
    pltpu.async_copy(pk_hbm.at[base + 3], ib.at[3], isem[3])
    pltpu.async_copy(x_hbm.at[ib.at[0, 1]], gbuf0, gsem[0])
    pltpu.async_copy(x_hbm.at[ib.at[1, 1]], gbuf1, gsem[1])

    def quad(q, _):
        j_base = 4 * q
        for u in range(4):
            j = j_base + u
            b = u % 2
            r = u
            rn = (u + 2) % 4
            pltpu.make_async_copy(x_hbm.at[ib.at[r, 1]], gbufs[b],
                                  gsem[b]).wait()
            scale(r, gbufs[b])
            pltpu.sync_copy(gbufs[b], acc.at[ib.at[r, 0]], add=True)

            @pl.when(j + 4 < cnt)
            def _():
                pltpu.async_copy(pk_hbm.at[base + j + 4], ib.at[r], isem[r])

            @pl.when(j + 2 < cnt)
            def _():
                pltpu.make_async_copy(pk_hbm.at[base + j + 2], ib.at[rn],
                                      isem[rn]).wait()
                pltpu.async_copy(x_hbm.at[ib.at[rn, 1]], gbufs[b], gsem[b])
        return 0
    lax.fori_loop(0, nq, quad, 0)
    plsc.subcore_barrier()

    for t in range(10):
        k = sid + NSUB * t
        if t < 9:
            pltpu.sync_copy(acc.at[pl.ds(k * 64, 64)], gbuf0)
            pltpu.sync_copy(gbuf0, out_hbm.at[cid, pl.ds(k * 64, 64)])
        else:
            @pl.when(k < NSLAB - 1)
            def _():
                pltpu.sync_copy(acc.at[pl.ds(k * 64, 64)], gbuf0)
                pltpu.sync_copy(gbuf0, out_hbm.at[cid, pl.ds(k * 64, 64)])
            @pl.when(k == NSLAB - 1)
            def _():
                pltpu.sync_copy(acc.at[pl.ds(k * 64, 16)],
                                gbuf0.at[pl.ds(0, 16)])
                pltpu.sync_copy(gbuf0.at[pl.ds(0, 16)],
                                out_hbm.at[cid, pl.ds(k * 64, 16)])


_spmm_partial = functools.partial(
    pl.kernel,
    mesh=plsc.VectorSubcoreMesh(core_axis_name="c", subcore_axis_name="s"),
    out_type=jax.ShapeDtypeStruct((NCORE, MPAD, FW), jnp.float32),
    scratch_types=[
        pltpu.VMEM((4, 3, ECHUNK), jnp.int32),
        pltpu.VMEM((ECHUNK, FW), jnp.float32),
        pltpu.VMEM((ECHUNK, FW), jnp.float32),
        pltpu.VMEM_SHARED((MV, FW), jnp.float32),
        pltpu.SemaphoreType.DMA,
        pltpu.SemaphoreType.DMA,
        pltpu.SemaphoreType.DMA,
        pltpu.SemaphoreType.DMA,
        pltpu.SemaphoreType.DMA,
        pltpu.SemaphoreType.DMA,
    ],
)(_spmm_body)


MB = 1024


def _combine_body(p_ref, a0_ref, o_ref):
    o_ref[...] = p_ref[0] + p_ref[1] - a0_ref[...]


def _combine(partials, a0):
    return pl.pallas_call(
        _combine_body,
        grid=(MPAD // MB,),
        in_specs=[
            pl.BlockSpec((NCORE, MB, FW), lambda i: (0, i, 0)),
            pl.BlockSpec((MB, FW), lambda i: (i, 0)),
        ],
        out_specs=pl.BlockSpec((MB, FW), lambda i: (i, 0)),
        out_shape=jax.ShapeDtypeStruct((MPAD, FW), jnp.float32),
    )(partials, a0)


def _matmul_body(q_ref, x1_ref, a0_ref, w_ref, o_ref):
    a0 = a0_ref[...]
    x1 = x1_ref[...]
    t2 = q_ref[0] + q_ref[1] - x1
    x2 = 2.0 * t2 - a0
    w = w_ref[...]
    for n in range(NBATCH):
        o_ref[n, :, :] = (
            jnp.dot(a0, w[0, n], preferred_element_type=jnp.float32)
            + jnp.dot(x1, w[1, n], preferred_element_type=jnp.float32)
            + jnp.dot(x2, w[2, n], preferred_element_type=jnp.float32)
        )


MMB = 1000


def _matmul(partials2, x1, a0, wn):
    return pl.pallas_call(
        _matmul_body,
        grid=(MV // MMB,),
        in_specs=[
            pl.BlockSpec((NCORE, MMB, FW), lambda i: (0, i, 0)),
            pl.BlockSpec((MMB, FW), lambda i: (i, 0)),
            pl.BlockSpec((MMB, FW), lambda i: (i, 0)),
            pl.BlockSpec((CHEB_K, NBATCH, FW, GOUT), lambda i: (0, 0, 0, 0)),
        ],
        out_specs=pl.BlockSpec((NBATCH, MMB, GOUT), lambda i: (0, i, 0)),
        out_shape=jax.ShapeDtypeStruct((NBATCH, MV, GOUT), jnp.float32),
    )(partials2, x1, a0, wn)


JB = 10000


def _gates_body(cv_ref, b_ref, c_ref, h_ref, cn_ref):
    cv = cv_ref[...] + b_ref[...][None, :, :]
    gi = jax.nn.sigmoid(cv[:, 0:32])
    gf = jax.nn.sigmoid(cv[:, 32:64])
    go = jax.nn.sigmoid(cv[:, 64:96])
    gg = jnp.tanh(cv[:, 96:128])
    cn = gf * c_ref[...] + gi * gg
    cn_ref[...] = cn
    h_ref[...] = go * jnp.tanh(cn)


def _gates(conv, bias_b, c_cur):
    return pl.pallas_call(
        _gates_body,
        grid=(1,),
        in_specs=[
            pl.BlockSpec((NBATCH, GOUT, JB), lambda i: (0, 0, 0)),
            pl.BlockSpec((GOUT, JB), lambda i: (0, 0)),
            pl.BlockSpec((NBATCH, 32, JB), lambda i: (0, 0, 0)),
        ],
        out_specs=[
            pl.BlockSpec((NBATCH, 32, JB), lambda i: (0, 0, 0)),
            pl.BlockSpec((NBATCH, 32, JB), lambda i: (0, 0, 0)),
        ],
        out_shape=[
            jax.ShapeDtypeStruct((NBATCH, 32, MV), jnp.float32),
            jax.ShapeDtypeStruct((NBATCH, 32, MV), jnp.float32),
        ],
    )(conv, bias_b, c_cur)


def kernel(input_tensor, h_cur, c_cur, L_vals, weight, bias, L_rows, L_cols):
    h_cur = h_cur.astype(jnp.float32)
    c_cur = c_cur.astype(jnp.float32)

    combined = jnp.concatenate([input_tensor, h_cur], axis=1)
    a0 = jnp.transpose(combined, (2, 1, 0)).reshape(MV, FW)
    a0 = jnp.pad(a0, ((0, MPAD - MV), (0, 0)))

    pad = NNZ_PAD - NNZV
    rows3 = jnp.concatenate(
        [L_rows, jnp.zeros((pad,), jnp.int32)]).reshape(NWORK, NCHUNK, ECHUNK)
    cols3 = jnp.concatenate(
        [L_cols, jnp.zeros((pad,), jnp.int32)]).reshape(NWORK, NCHUNK, ECHUNK)
    vbits3 = lax.bitcast_convert_type(jnp.concatenate(
        [L_vals, jnp.zeros((pad,), jnp.float32)]), jnp.int32
        ).reshape(NWORK, NCHUNK, ECHUNK)
    pk = jnp.stack([rows3, cols3, vbits3], axis=2).reshape(
        NWORK * NCHUNK, 3, ECHUNK)

    wr = weight.reshape(FTOT, CHEB_K, GOUT).transpose(1, 0, 2)
    wz = jnp.zeros_like(wr)
    wn0 = jnp.stack([wr, wz], axis=2).reshape(CHEB_K, FW, GOUT)
    wn1 = jnp.stack([wz, wr], axis=2).reshape(CHEB_K, FW, GOUT)
    wn = jnp.stack([wn0, wn1], axis=1)

    p1 = _spmm_partial(a0, pk)
    x1 = _combine(p1, a0)
    p2 = _spmm_partial(x1, pk)
    mat = _matmul(p2, x1, a0, wn)

    conv = mat.reshape(NBATCH, GOUT, MV)
    bias_b = jnp.broadcast_to(bias.reshape(GOUT, 1), (GOUT, JB))
    h_next, c_next = _gates(conv, bias_b, c_cur)
    return (h_next, c_next)
